# baseline (speedup 1.0000x reference)
import functools

import jax
import jax.numpy as jnp
from jax import lax
from jax.experimental import pallas as pl
from jax.experimental.pallas import tpu as pltpu
from jax.experimental.pallas import tpu_sc as plsc

N = 16777216
NBINS = 256
NW = 32
PER_W = N // NW
CHUNK = 32768
NCHUNK = PER_W // CHUNK
HSTRIDE = NBINS + 1
UNROLL = 12
LANES = 16

_mesh = plsc.VectorSubcoreMesh(core_axis_name="c", subcore_axis_name="s")


@functools.partial(
    pl.kernel,
    mesh=_mesh,
    out_type=jax.ShapeDtypeStruct((NW, NBINS), jnp.float32),
    compiler_params=pltpu.CompilerParams(needs_layout_passes=False),
    scratch_types=[
        pltpu.VMEM((CHUNK,), jnp.float32),
        pltpu.VMEM((CHUNK,), jnp.float32),
        pltpu.VMEM((LANES * HSTRIDE,), jnp.float32),
        pltpu.VMEM((NBINS,), jnp.float32),
        pltpu.SemaphoreType.DMA,
        pltpu.SemaphoreType.DMA,
    ],
)
def _hist_kernel(x_hbm, out_hbm, buf0, buf1, hist, hrow, sem0, sem1):
    cid = lax.axis_index("c")
    sid = lax.axis_index("s")
    wid = sid * 2 + cid
    base = wid * PER_W

    zeros = jnp.zeros((LANES,), jnp.float32)
    ones = jnp.ones((LANES,), jnp.float32)
    lane_base = lax.iota(jnp.int32, LANES) * HSTRIDE

    def zero_body(j, _):
        hist[pl.ds(j * LANES, LANES)] = zeros
        return 0
    lax.fori_loop(0, HSTRIDE, zero_body, 0)

    def process(buf):
        @plsc.parallel_loop(0, CHUNK // LANES, unroll=UNROLL)
        def _(j):
            v = buf[pl.ds(j * LANES, LANES)]
            v = jnp.maximum(v, -4.0)
            t = (v + 4.0) * 32.0
            t = jnp.minimum(t, 255.0)
            idx = t.astype(jnp.int32)
            plsc.addupdate_scatter(hist, [idx + lane_base], ones)

    bufs = (buf0, buf1)
    sems = (sem0, sem1)

    pltpu.async_copy(x_hbm.at[pl.ds(base, CHUNK)], buf0, sem0)

    def pair_body(p, _):
        for b in range(2):
            c = p * 2 + b
            nxt = c + 1

            @pl.when(nxt < NCHUNK)
            def _():
                pltpu.async_copy(
                    x_hbm.at[pl.ds(base + nxt * CHUNK, CHUNK)],
                    bufs[(b + 1) % 2], sems[(b + 1) % 2])

            pltpu.make_async_copy(
                x_hbm.at[pl.ds(base + c * CHUNK, CHUNK)],
                bufs[b], sems[b]).wait()
            process(bufs[b])
        return 0
    lax.fori_loop(0, NCHUNK // 2, pair_body, 0)

    for g in range(NBINS // LANES):
        acc = hist[pl.ds(g * LANES, LANES)]
        for lane in range(1, LANES):
            acc = acc + hist[pl.ds(lane * HSTRIDE + g * LANES, LANES)]
        hrow[pl.ds(g * LANES, LANES)] = acc

    pltpu.sync_copy(hrow, out_hbm.at[wid])


def kernel(x):
    partials = _hist_kernel(x)
    return jnp.sum(partials, axis=0)


# speedup vs baseline: 1.0421x; 1.0421x over previous
import functools

import jax
import jax.numpy as jnp
from jax import lax
from jax.experimental import pallas as pl
from jax.experimental.pallas import tpu as pltpu
from jax.experimental.pallas import tpu_sc as plsc

N = 16777216
NBINS = 256
NW = 32
PER_W = N // NW
CHUNK = 32768
NCHUNK = PER_W // CHUNK
HSTRIDE = NBINS + 1
UNROLL = 8
LANES = 16

_mesh = plsc.VectorSubcoreMesh(core_axis_name="c", subcore_axis_name="s")


@functools.partial(
    pl.kernel,
    mesh=_mesh,
    out_type=jax.ShapeDtypeStruct((NW, NBINS), jnp.float32),
    compiler_params=pltpu.CompilerParams(
        needs_layout_passes=False,
        disable_bounds_checks=True,
        disable_semaphore_checks=True,
    ),
    scratch_types=[
        pltpu.VMEM((CHUNK,), jnp.float32),
        pltpu.VMEM((CHUNK,), jnp.float32),
        pltpu.VMEM((LANES * HSTRIDE,), jnp.float32),
        pltpu.VMEM((NBINS,), jnp.float32),
        pltpu.SemaphoreType.DMA,
        pltpu.SemaphoreType.DMA,
    ],
)
def _hist_kernel(x_hbm, out_hbm, buf0, buf1, hist, hrow, sem0, sem1):
    cid = lax.axis_index("c")
    sid = lax.axis_index("s")
    wid = sid * 2 + cid
    base = wid * PER_W

    zeros = jnp.zeros((LANES,), jnp.float32)
    ones = jnp.ones((LANES,), jnp.float32)
    lane_base = lax.iota(jnp.int32, LANES) * HSTRIDE

    def zero_body(j, _):
        hist[pl.ds(j * LANES, LANES)] = zeros
        return 0
    lax.fori_loop(0, HSTRIDE, zero_body, 0)

    def process(buf):
        @plsc.parallel_loop(0, CHUNK // LANES, unroll=UNROLL)
        def _(j):
            v = buf[pl.ds(j * LANES, LANES)]
            v = jnp.maximum(v, -4.0)
            t = (v + 4.0) * 32.0
            t = jnp.minimum(t, 255.0)
            idx = t.astype(jnp.int32)
            plsc.addupdate_scatter(hist, [idx + lane_base], ones)

    bufs = (buf0, buf1)
    sems = (sem0, sem1)

    pltpu.async_copy(x_hbm.at[pl.ds(base, CHUNK)], buf0, sem0)

    def pair_body(p, _):
        for b in range(2):
            c = p * 2 + b
            nxt = c + 1

            @pl.when(nxt < NCHUNK)
            def _():
                pltpu.async_copy(
                    x_hbm.at[pl.ds(base + nxt * CHUNK, CHUNK)],
                    bufs[(b + 1) % 2], sems[(b + 1) % 2])

            pltpu.make_async_copy(
                x_hbm.at[pl.ds(base + c * CHUNK, CHUNK)],
                bufs[b], sems[b]).wait()
            process(bufs[b])
        return 0
    lax.fori_loop(0, NCHUNK // 2, pair_body, 0)

    for g in range(NBINS // LANES):
        acc = hist[pl.ds(g * LANES, LANES)]
        for lane in range(1, LANES):
            acc = acc + hist[pl.ds(lane * HSTRIDE + g * LANES, LANES)]
        hrow[pl.ds(g * LANES, LANES)] = acc

    pltpu.sync_copy(hrow, out_hbm.at[wid])


def kernel(x):
    partials = _hist_kernel(x)
    return jnp.sum(partials, axis=0)
